# baseline (speedup 1.0000x reference)
import functools

import jax
import jax.numpy as jnp
import numpy as np
from jax import lax
from jax.experimental import pallas as pl
from jax.experimental.pallas import tpu as pltpu
from jax.experimental.pallas import tpu_sc as plsc

_B, _T, _C, _H, _W = 4, 16, 256, 56, 56
_HW = _H * _W
_R = _B * _T * _C
_K = 4
_NC, _NS = 2, 16
_NW = _NC * _NS
_CH = 8


def _build_indices():
    r = np.arange(_R, dtype=np.int64)
    t = (r // _C) % _T
    c = r % _C
    fwd = (c % 3 == 0) & (c != _C - 1)
    back = c % 3 == 1
    zero = (fwd & (t < _T - _K)) | (back & (t < _K))
    src = np.where(back, r - _K * _C, r)

    def _split_pad(arrs, n):
        per_w = ((-(-n // _NW) + _CH - 1) // _CH) * _CH
        pos = np.array_split(np.arange(n), _NW)
        out = []
        for a in arrs:
            rows = []
            for p in pos:
                g = a[p]
                g = np.concatenate([g, np.repeat(g[-1:], per_w - g.shape[0])])
                rows.append(g)
            out.append(
                np.stack(rows).reshape(_NW, per_w // _CH, _CH).astype(np.int32)
            )
        return out

    copy_mask = ~zero
    csrc, cdst = _split_pad(
        [src[copy_mask], r[copy_mask]], int(copy_mask.sum())
    )
    (zdst,) = _split_pad([r[zero]], int(zero.sum()))
    return np.concatenate([csrc, cdst, zdst], axis=1), csrc.shape[1], zdst.shape[1]


_IDX, _NCCH, _NZCH = _build_indices()


def _sc_body(x_hbm, idx_hbm, out_hbm, idx_v, buf_a, buf_b, zbuf,
             gsa, gsb, ssa, ssb, zs):
    wid = lax.axis_index("s") * _NC + lax.axis_index("c")
    pltpu.sync_copy(idx_hbm.at[wid], idx_v)

    def zinit(q, carry):
        row = q // (_HW // 16)
        col = 16 * (q % (_HW // 16))
        zbuf[row, pl.ds(col, 16)] = jnp.zeros((16,), jnp.float32)
        return carry

    lax.fori_loop(0, _CH * (_HW // 16), zinit, 0)

    def zfire(j, carry):
        pltpu.make_async_copy(
            zbuf, out_hbm.at[idx_v.at[2 * _NCCH + j]], zs).start()
        return carry

    lax.fori_loop(0, _NZCH, zfire, 0)

    def cbody(q, carry):
        @pl.when(q > 0)
        def _():
            pltpu.make_async_copy(buf_a, out_hbm.at[pl.ds(0, _CH)], ssa).wait()

        ga = pltpu.make_async_copy(x_hbm.at[idx_v.at[2 * q]], buf_a, gsa)
        ga.start()

        @pl.when(q > 0)
        def _():
            pltpu.make_async_copy(buf_b, out_hbm.at[pl.ds(0, _CH)], ssb).wait()

        gb = pltpu.make_async_copy(x_hbm.at[idx_v.at[2 * q + 1]], buf_b, gsb)
        gb.start()

        ga.wait()
        pltpu.make_async_copy(
            buf_a, out_hbm.at[idx_v.at[_NCCH + 2 * q]], ssa).start()
        gb.wait()
        pltpu.make_async_copy(
            buf_b, out_hbm.at[idx_v.at[_NCCH + 2 * q + 1]], ssb).start()
        return carry

    lax.fori_loop(0, _NCCH // 2, cbody, 0)
    pltpu.make_async_copy(buf_a, out_hbm.at[pl.ds(0, _CH)], ssa).wait()
    gt = pltpu.make_async_copy(x_hbm.at[idx_v.at[_NCCH - 1]], buf_a, gsa)
    gt.start()
    gt.wait()
    pltpu.make_async_copy(
        buf_a, out_hbm.at[idx_v.at[2 * _NCCH - 1]], ssa).start()

    pltpu.make_async_copy(buf_a, out_hbm.at[pl.ds(0, _CH)], ssa).wait()
    pltpu.make_async_copy(buf_b, out_hbm.at[pl.ds(0, _CH)], ssb).wait()

    def zdrain(j, carry):
        pltpu.make_async_copy(zbuf, out_hbm.at[pl.ds(0, _CH)], zs).wait()
        return carry

    lax.fori_loop(0, _NZCH, zdrain, 0)


@functools.lru_cache(maxsize=1)
def _get_sc_call():
    return functools.partial(
        pl.kernel,
        out_type=jax.ShapeDtypeStruct((_R, _HW), jnp.float32),
        mesh=plsc.VectorSubcoreMesh(
            core_axis_name="c", subcore_axis_name="s",
            num_cores=_NC, num_subcores=_NS,
        ),
        scratch_types=[
            pltpu.VMEM((2 * _NCCH + _NZCH, _CH), jnp.int32),
            pltpu.VMEM((_CH, _HW), jnp.float32),
            pltpu.VMEM((_CH, _HW), jnp.float32),
            pltpu.VMEM((_CH, _HW), jnp.float32),
            pltpu.SemaphoreType.DMA,
            pltpu.SemaphoreType.DMA,
            pltpu.SemaphoreType.DMA,
            pltpu.SemaphoreType.DMA,
            pltpu.SemaphoreType.DMA,
        ],
        compiler_params=pltpu.CompilerParams(use_tc_tiling_on_sc=False),
    )(_sc_body)


def kernel(x, shift_factor, elements):
    del shift_factor, elements
    x2 = x.reshape(_R, _HW)
    out2 = _get_sc_call()(x2, jnp.asarray(_IDX))
    return out2.reshape(_B, _T, _C, _H, _W)


# speedup vs baseline: 1.3886x; 1.3886x over previous
import functools

import jax
import jax.numpy as jnp
from jax import lax
from jax.experimental import pallas as pl
from jax.experimental.pallas import tpu as pltpu
from jax.experimental.pallas import tpu_sc as plsc

_B, _T, _C, _H, _W = 4, 16, 256, 56, 56
_K = 4
_NC, _NS = 2, 16
_NBLK = 42


def _sc_body(x_hbm, zrow_hbm, out_hbm, bb0, bb1, zbuf, gsa, gsb, ssa, ssb):
    i32 = jnp.int32
    wid = lax.axis_index("s") * _NC + lax.axis_index("c")
    t = wid % _T
    b1 = wid // _T

    pltpu.sync_copy(zrow_hbm.at[pl.ds(0, 1)], zbuf)

    def sel(m):
        hi = (jnp.asarray(m) >= _NBLK).astype(i32)
        return 6 * (m - _NBLK * hi), b1 + 2 * hi

    def wdummy(bb, ss):
        return pltpu.make_async_copy(bb, out_hbm.at[0, 0, pl.ds(0, 6)], ss)

    def block_write(bb, ss, m):
        c0, b = sel(m)
        pltpu.make_async_copy(bb, out_hbm.at[b, t, pl.ds(c0, 6)], ss).start()

    def plane_in(b, tt, c, dst_ref, gs):
        return pltpu.make_async_copy(x_hbm.at[b, tt, pl.ds(c, 1)], dst_ref, gs)

    def body_a(bb, gs, ss, m, first):
        @pl.when(jnp.logical_not(first))
        def _():
            wdummy(bb, ss).wait()

        c0, b = sel(m)
        g0 = plane_in(b, t, c0 + 2, bb.at[pl.ds(2, 1)], gs)
        g0.start()
        g1 = plane_in(b, t, c0 + 5, bb.at[pl.ds(5, 1)], gs)
        g1.start()
        g0.wait()
        g1.wait()
        block_write(bb, ss, m)

    def body_b(bb, gs, ss, m, first):
        @pl.when(jnp.logical_not(first))
        def _():
            wdummy(bb, ss).wait()

        c0, b = sel(m)
        gs_ = [
            plane_in(b, t - _K, c0 + 1, bb.at[pl.ds(1, 1)], gs),
            plane_in(b, t - _K, c0 + 4, bb.at[pl.ds(4, 1)], gs),
            plane_in(b, t, c0 + 2, bb.at[pl.ds(2, 1)], gs),
            plane_in(b, t, c0 + 5, bb.at[pl.ds(5, 1)], gs),
        ]
        for g in gs_:
            g.start()
        for g in gs_:
            g.wait()
        block_write(bb, ss, m)

    def body_c(bb, gs, ss, m, first):
        @pl.when(jnp.logical_not(first))
        def _():
            wdummy(bb, ss).wait()

        c0, b = sel(m)
        gblk = pltpu.make_async_copy(x_hbm.at[b, t, pl.ds(c0, 6)], bb, gs)
        gblk.start()
        gblk.wait()
        g1 = plane_in(b, t - _K, c0 + 1, bb.at[pl.ds(1, 1)], gs)
        g1.start()
        g2 = plane_in(b, t - _K, c0 + 4, bb.at[pl.ds(4, 1)], gs)
        g2.start()
        g1.wait()
        g2.wait()
        block_write(bb, ss, m)

    def run_blocks(body):
        def loop(q, carry):
            body(bb0, gsa, ssa, 2 * q, q == 0)
            body(bb1, gsb, ssb, 2 * q + 1, q == 0)
            return carry

        lax.fori_loop(0, _NBLK, loop, 0)
        wdummy(bb0, ssa).wait()
        wdummy(bb1, ssb).wait()

    def copy_plane(b, ts, cs, td, cd, n):
        g = pltpu.make_async_copy(
            x_hbm.at[b, ts, pl.ds(cs, n)], bb0.at[pl.ds(0, n)], gsa)
        g.start()
        g.wait()
        s = pltpu.make_async_copy(
            bb0.at[pl.ds(0, n)], out_hbm.at[b, td, pl.ds(cd, n)], ssa)
        s.start()
        s.wait()

    def zero_plane(b, c):
        s = pltpu.make_async_copy(zbuf, out_hbm.at[b, t, pl.ds(c, 1)], ssa)
        s.start()
        s.wait()

    @pl.when(t < _K)
    def _bucket_a():
        for p in (0, 1):
            pltpu.sync_copy(zrow_hbm, bb0.at[pl.ds(3 * p, 2)])
            pltpu.sync_copy(zrow_hbm, bb1.at[pl.ds(3 * p, 2)])
        run_blocks(body_a)
        for b in (b1, b1 + 2):
            zero_plane(b, 252)
            zero_plane(b, 253)
            copy_plane(b, t, 254, t, 254, 2)

    @pl.when((t >= _K) & (t < _T - _K))
    def _bucket_b():
        for p in (0, 3):
            pltpu.sync_copy(zrow_hbm.at[pl.ds(0, 1)], bb0.at[pl.ds(p, 1)])
            pltpu.sync_copy(zrow_hbm.at[pl.ds(0, 1)], bb1.at[pl.ds(p, 1)])
        run_blocks(body_b)
        for b in (b1, b1 + 2):
            zero_plane(b, 252)
            copy_plane(b, t - _K, 253, t, 253, 1)
            copy_plane(b, t, 254, t, 254, 2)

    @pl.when(t >= _T - _K)
    def _bucket_c():
        run_blocks(body_c)
        for b in (b1, b1 + 2):
            copy_plane(b, t, 252, t, 252, 1)
            copy_plane(b, t - _K, 253, t, 253, 1)
            copy_plane(b, t, 254, t, 254, 2)


@functools.lru_cache(maxsize=1)
def _get_sc_call():
    return functools.partial(
        pl.kernel,
        out_type=jax.ShapeDtypeStruct((_B, _T, _C, _H, _W), jnp.float32),
        mesh=plsc.VectorSubcoreMesh(
            core_axis_name="c", subcore_axis_name="s",
            num_cores=_NC, num_subcores=_NS,
        ),
        scratch_types=[
            pltpu.VMEM((6, _H, _W), jnp.float32),
            pltpu.VMEM((6, _H, _W), jnp.float32),
            pltpu.VMEM((1, _H, _W), jnp.float32),
            pltpu.SemaphoreType.DMA,
            pltpu.SemaphoreType.DMA,
            pltpu.SemaphoreType.DMA,
            pltpu.SemaphoreType.DMA,
        ],
        compiler_params=pltpu.CompilerParams(use_tc_tiling_on_sc=True),
    )(_sc_body)


def kernel(x, shift_factor, elements):
    del shift_factor, elements
    zrow = jnp.zeros((2, _H, _W), jnp.float32)
    return _get_sc_call()(x, zrow)


# speedup vs baseline: 2.0906x; 1.5055x over previous
import functools

import jax
import jax.numpy as jnp
from jax import lax
from jax.experimental import pallas as pl
from jax.experimental.pallas import tpu as pltpu
from jax.experimental.pallas import tpu_sc as plsc

_B, _T, _C, _H, _W = 4, 16, 256, 56, 56
_R = _B * _T * _C
_K = 4
_NC, _NS = 2, 16
_SLAB = 2 * _T * _C


def _sc_body(x_hbm, zrow_hbm, out_hbm, buf, zbuf,
             gs0, gs1, gs2, gs3, ss0, ss1, ss2, ss3, zs):
    i32 = jnp.int32
    wid = lax.axis_index("s") * _NC + lax.axis_index("c")
    t = wid % _T
    base1 = (wid // _T) * (_T * _C) + t * _C
    gsems = (gs0, gs1, gs2, gs3)
    ssems = (ss0, ss1, ss2, ss3)

    pltpu.sync_copy(zrow_hbm, zbuf)

    def sel(j):
        hi = (jnp.asarray(j) >= 85).astype(i32)
        return j - 85 * hi, base1 + _SLAB * hi

    def slot(s, L):
        return buf.at[pl.ds(s * L, L)]

    def ring4(n4, L, src_row, dst_row):
        dummy = out_hbm.at[pl.ds(0, L)]

        def body(q, carry):
            base = 4 * q

            for s in range(4):
                @pl.when(q > 0)
                def _(s=s):
                    pltpu.make_async_copy(slot(s, L), dummy, ssems[s]).wait()

                pltpu.make_async_copy(
                    x_hbm.at[pl.ds(src_row(base + s), L)],
                    slot(s, L), gsems[s]).start()
            for s in range(4):
                pltpu.make_async_copy(
                    x_hbm.at[pl.ds(0, L)], slot(s, L), gsems[s]).wait()
                pltpu.make_async_copy(
                    slot(s, L), out_hbm.at[pl.ds(dst_row(base + s), L)],
                    ssems[s]).start()
            return carry

        lax.fori_loop(0, n4, body, 0)
        for s in range(4):
            pltpu.make_async_copy(slot(s, L), dummy, ssems[s]).wait()

    def single(src, dst, L=1):
        g = pltpu.make_async_copy(x_hbm.at[pl.ds(src, L)], slot(0, L), gs0)
        g.start()
        g.wait()
        s = pltpu.make_async_copy(slot(0, L), out_hbm.at[pl.ds(dst, L)], ss0)
        s.start()
        s.wait()

    def ident_row(j):
        jj, base = sel(j)
        return base + 3 * jj + 2

    def shift_dst(j):
        jj, base = sel(j)
        return base + 3 * jj + 1

    def shift_src(j):
        return shift_dst(j) - _K * _C

    @pl.when(t < _K)
    def _bucket_a():
        def zfire(j, carry):
            jj, base = sel(j)
            pltpu.make_async_copy(
                zbuf, out_hbm.at[pl.ds(base + 3 * jj, 2)], zs).start()
            return carry

        lax.fori_loop(0, 170, zfire, 0)
        ring4(42, 1, ident_row, ident_row)
        single(ident_row(168), ident_row(168))
        single(ident_row(169), ident_row(169))
        single(base1 + 255, base1 + 255)
        single(base1 + _SLAB + 255, base1 + _SLAB + 255)

        def zdrain(j, carry):
            pltpu.make_async_copy(zbuf, out_hbm.at[pl.ds(0, 2)], zs).wait()
            return carry

        lax.fori_loop(0, 170, zdrain, 0)

    @pl.when((t >= _K) & (t < _T - _K))
    def _bucket_b():
        def zfire(j, carry):
            jj, base = sel(j)
            pltpu.make_async_copy(
                zbuf.at[pl.ds(0, 1)],
                out_hbm.at[pl.ds(base + 3 * jj, 1)], zs).start()
            return carry

        lax.fori_loop(0, 170, zfire, 0)
        ring4(42, 1, shift_src, shift_dst)
        single(shift_src(168), shift_dst(168))
        single(shift_src(169), shift_dst(169))
        ring4(42, 1, ident_row, ident_row)
        single(ident_row(168), ident_row(168))
        single(ident_row(169), ident_row(169))
        single(base1 + 255, base1 + 255)
        single(base1 + _SLAB + 255, base1 + _SLAB + 255)

        def zdrain(j, carry):
            pltpu.make_async_copy(
                zbuf.at[pl.ds(0, 1)], out_hbm.at[pl.ds(0, 1)], zs).wait()
            return carry

        lax.fori_loop(0, 170, zdrain, 0)

    @pl.when(t >= _T - _K)
    def _bucket_c():
        def pair_row(j):
            jj, base = sel(j)
            c = jnp.where(jj == 84, 254, 3 * jj + 2)
            return base + c

        ring4(42, 1, shift_src, shift_dst)
        single(shift_src(168), shift_dst(168))
        single(shift_src(169), shift_dst(169))
        ring4(42, 2, pair_row, pair_row)
        single(pair_row(168), pair_row(168), L=2)
        single(pair_row(169), pair_row(169), L=2)
        single(base1, base1)
        single(base1 + _SLAB, base1 + _SLAB)


@functools.lru_cache(maxsize=1)
def _get_sc_call():
    return functools.partial(
        pl.kernel,
        out_type=jax.ShapeDtypeStruct((_R, _H, _W), jnp.float32),
        mesh=plsc.VectorSubcoreMesh(
            core_axis_name="c", subcore_axis_name="s",
            num_cores=_NC, num_subcores=_NS,
        ),
        scratch_types=[
            pltpu.VMEM((8, _H, _W), jnp.float32),
            pltpu.VMEM((2, _H, _W), jnp.float32),
            pltpu.SemaphoreType.DMA,
            pltpu.SemaphoreType.DMA,
            pltpu.SemaphoreType.DMA,
            pltpu.SemaphoreType.DMA,
            pltpu.SemaphoreType.DMA,
            pltpu.SemaphoreType.DMA,
            pltpu.SemaphoreType.DMA,
            pltpu.SemaphoreType.DMA,
            pltpu.SemaphoreType.DMA,
        ],
        compiler_params=pltpu.CompilerParams(use_tc_tiling_on_sc=True),
    )(_sc_body)


def kernel(x, shift_factor, elements):
    del shift_factor, elements
    x3 = x.reshape(_R, _H, _W)
    zrow = jnp.zeros((2, _H, _W), jnp.float32)
    out3 = _get_sc_call()(x3, zrow)
    return out3.reshape(_B, _T, _C, _H, _W)
